# baseline (speedup 1.0000x reference)
import jax
import jax.numpy as jnp
from jax import lax
from jax.experimental import pallas as pl
from jax.experimental.pallas import tpu as pltpu
from jax.experimental.pallas import tpu_sc as plsc

_E = 64
_N = 32768
_NC = 2
_NS = 16
_NW = _NC * _NS
_TPW = _N // _NW
_L = 16
_CHUNK = 256
_NCHUNK = _TPW // _CHUNK
_TPB = 8
_NV = _E // _L


def _sc_body(x_hbm, mask_hbm, cnt_hbm, psum_hbm,
             xv0, xv1, mv, cntv, psv, sem0, sem1):
    wid = lax.axis_index("s") * _NC + lax.axis_index("c")
    base = wid * _TPW
    pltpu.sync_copy(mask_hbm.at[pl.ds(base, _TPW)], mv.at[pl.ds(0, _TPW)])

    ios = [lax.iota(jnp.int32, _L) + k * _L for k in range(_NV)]

    def token(t, xv, moff, acc):
        v = [xv[t, pl.ds(k * _L, _L)] for k in range(_NV)]
        m = jnp.maximum(jnp.maximum(v[0], v[1]), jnp.maximum(v[2], v[3]))
        smax = jnp.max(m)
        ids = [jnp.where(v[k] == smax, ios[k], _E) for k in range(_NV)]
        imin = jnp.minimum(jnp.minimum(ids[0], ids[1]),
                           jnp.minimum(ids[2], ids[3]))
        fidx = jnp.min(imin)
        valid = mv[pl.ds(moff + t, _L)][0] == 1
        fidx = jnp.where(valid, fidx, _E)
        cs, ps = acc
        cs = tuple(cs[k] + jnp.where(ios[k] == fidx, 1.0, 0.0)
                   for k in range(_NV))
        ps = tuple(ps[k] + jnp.where(ios[k] == fidx, smax, 0.0)
                   for k in range(_NV))
        return cs, ps

    acc = (tuple(jnp.zeros((_L,), jnp.float32) for _ in range(_NV)),
           tuple(jnp.zeros((_L,), jnp.float32) for _ in range(_NV)))

    bufs = (xv0, xv1)
    sems = (sem0, sem1)
    copies = [None] * _NCHUNK
    copies[0] = pltpu.async_copy(x_hbm.at[pl.ds(base, _CHUNK)], xv0, sem0)

    for c in range(_NCHUNK):
        copies[c].wait()
        if c + 1 < _NCHUNK:
            copies[c + 1] = pltpu.async_copy(
                x_hbm.at[pl.ds(base + (c + 1) * _CHUNK, _CHUNK)],
                bufs[(c + 1) % 2], sems[(c + 1) % 2])
        xv = bufs[c % 2]

        @pl.loop(0, _CHUNK // _TPB, init_carry=acc)
        def _blk(b, carry):
            t0 = b * _TPB
            for j in range(_TPB):
                carry = token(t0 + j, xv, c * _CHUNK, carry)
            return carry

        acc = _blk

    cs, ps = acc
    for k in range(_NV):
        cntv[pl.ds(k * _L, _L)] = cs[k]
        psv[pl.ds(k * _L, _L)] = ps[k]
    pltpu.sync_copy(cntv, cnt_hbm.at[wid])
    pltpu.sync_copy(psv, psum_hbm.at[wid])


_sc_call = pl.kernel(
    _sc_body,
    out_type=[
        jax.ShapeDtypeStruct((_NW, _E), jnp.float32),
        jax.ShapeDtypeStruct((_NW, _E), jnp.float32),
    ],
    mesh=plsc.VectorSubcoreMesh(core_axis_name="c", subcore_axis_name="s",
                                num_cores=_NC, num_subcores=_NS),
    compiler_params=pltpu.CompilerParams(needs_layout_passes=False),
    scratch_types=[
        pltpu.VMEM((_CHUNK, _E), jnp.float32),
        pltpu.VMEM((_CHUNK, _E), jnp.float32),
        pltpu.VMEM((_TPW + _L,), jnp.int32),
        pltpu.VMEM((_E,), jnp.float32),
        pltpu.VMEM((_E,), jnp.float32),
        pltpu.SemaphoreType.DMA,
        pltpu.SemaphoreType.DMA,
    ],
)


def _combine_body(cnt_ref, ps_ref, out_ref):
    cnt = jnp.sum(cnt_ref[...], axis=0, keepdims=True)
    ps = jnp.sum(ps_ref[...], axis=0, keepdims=True)
    nv = jnp.sum(cnt)
    loss = _E * jnp.sum(cnt * ps) / (nv * nv * nv)
    out_ref[...] = jnp.full((1, 1), loss, dtype=jnp.float32)


def kernel(x, mask):
    xr = x.reshape(_N, _E)
    mr = mask.reshape(_N)
    cnt, psum = _sc_call(xr, mr)
    out = pl.pallas_call(
        _combine_body,
        out_shape=jax.ShapeDtypeStruct((1, 1), jnp.float32),
    )(cnt, psum)
    return out[0, 0]


# speedup vs baseline: 1.1278x; 1.1278x over previous
import jax
import jax.numpy as jnp
from jax import lax
from jax.experimental import pallas as pl
from jax.experimental.pallas import tpu as pltpu
from jax.experimental.pallas import tpu_sc as plsc

_E = 64
_N = 32768
_NC = 2
_NS = 16
_NW = _NC * _NS
_TPW = _N // _NW
_L = 16
_CHUNK = 256
_NCHUNK = _TPW // _CHUNK
_TPB = 2
_NV = _E // _L


def _sc_body(x_hbm, mask_hbm, cnt_hbm, psum_hbm,
             xv0, xv1, mv, cntv, psv, sem0, sem1):
    wid = lax.axis_index("s") * _NC + lax.axis_index("c")
    base = wid * _TPW
    pltpu.sync_copy(mask_hbm.at[pl.ds(base, _TPW)], mv.at[pl.ds(0, _TPW)])

    ios = [lax.iota(jnp.int32, _L) + k * _L for k in range(_NV)]

    def token(t, xv, moff, acc):
        v = [xv[t, pl.ds(k * _L, _L)] for k in range(_NV)]
        m = jnp.maximum(jnp.maximum(v[0], v[1]), jnp.maximum(v[2], v[3]))
        smax = jnp.max(m)
        ids = [jnp.where(v[k] == smax, ios[k], _E) for k in range(_NV)]
        imin = jnp.minimum(jnp.minimum(ids[0], ids[1]),
                           jnp.minimum(ids[2], ids[3]))
        fidx = jnp.min(imin)
        valid = mv[pl.ds(moff + t, _L)][0] == 1
        fidx = jnp.where(valid, fidx, _E)
        cs, ps = acc
        cs = tuple(cs[k] + jnp.where(ios[k] == fidx, 1.0, 0.0)
                   for k in range(_NV))
        ps = tuple(ps[k] + jnp.where(ios[k] == fidx, smax, 0.0)
                   for k in range(_NV))
        return cs, ps

    acc = (tuple(jnp.zeros((_L,), jnp.float32) for _ in range(_NV)),
           tuple(jnp.zeros((_L,), jnp.float32) for _ in range(_NV)))

    bufs = (xv0, xv1)
    sems = (sem0, sem1)
    copies = [None] * _NCHUNK
    copies[0] = pltpu.async_copy(x_hbm.at[pl.ds(base, _CHUNK)], xv0, sem0)

    for c in range(_NCHUNK):
        copies[c].wait()
        if c + 1 < _NCHUNK:
            copies[c + 1] = pltpu.async_copy(
                x_hbm.at[pl.ds(base + (c + 1) * _CHUNK, _CHUNK)],
                bufs[(c + 1) % 2], sems[(c + 1) % 2])
        xv = bufs[c % 2]

        @pl.loop(0, _CHUNK // _TPB, init_carry=acc)
        def _blk(b, carry):
            t0 = b * _TPB
            for j in range(_TPB):
                carry = token(t0 + j, xv, c * _CHUNK, carry)
            return carry

        acc = _blk

    cs, ps = acc
    for k in range(_NV):
        cntv[pl.ds(k * _L, _L)] = cs[k]
        psv[pl.ds(k * _L, _L)] = ps[k]
    pltpu.sync_copy(cntv, cnt_hbm.at[wid])
    pltpu.sync_copy(psv, psum_hbm.at[wid])


_sc_call = pl.kernel(
    _sc_body,
    out_type=[
        jax.ShapeDtypeStruct((_NW, _E), jnp.float32),
        jax.ShapeDtypeStruct((_NW, _E), jnp.float32),
    ],
    mesh=plsc.VectorSubcoreMesh(core_axis_name="c", subcore_axis_name="s",
                                num_cores=_NC, num_subcores=_NS),
    compiler_params=pltpu.CompilerParams(needs_layout_passes=False),
    scratch_types=[
        pltpu.VMEM((_CHUNK, _E), jnp.float32),
        pltpu.VMEM((_CHUNK, _E), jnp.float32),
        pltpu.VMEM((_TPW + _L,), jnp.int32),
        pltpu.VMEM((_E,), jnp.float32),
        pltpu.VMEM((_E,), jnp.float32),
        pltpu.SemaphoreType.DMA,
        pltpu.SemaphoreType.DMA,
    ],
)


def _combine_body(cnt_ref, ps_ref, out_ref):
    cnt = jnp.sum(cnt_ref[...], axis=0, keepdims=True)
    ps = jnp.sum(ps_ref[...], axis=0, keepdims=True)
    nv = jnp.sum(cnt)
    loss = _E * jnp.sum(cnt * ps) / (nv * nv * nv)
    out_ref[...] = jnp.full((1, 1), loss, dtype=jnp.float32)


def kernel(x, mask):
    xr = x.reshape(_N, _E)
    mr = mask.reshape(_N)
    cnt, psum = _sc_call(xr, mr)
    out = pl.pallas_call(
        _combine_body,
        out_shape=jax.ShapeDtypeStruct((1, 1), jnp.float32),
    )(cnt, psum)
    return out[0, 0]
